# baseline (speedup 1.0000x reference)
import functools

import jax
import jax.numpy as jnp
from jax import lax
from jax.experimental import pallas as pl
from jax.experimental.pallas import tpu as pltpu
from jax.experimental.pallas import tpu_sc as plsc

N_NODES = 10000
N_EDGES = 320000
D_FEAT = 128
D_OUT = D_FEAT + 3

NC = 2
NS = 16
NW = NC * NS
CHUNK = 128
NCHUNKS_TOTAL = N_EDGES // CHUNK
NGROUPS = CHUNK // 16


def _sc_kernel(pos_hbm, feat_hbm, src_hbm, dst_hbm, out_hbm,
               posv, srcv, dstv, fbuf, obuf, sem):
    wid = lax.axis_index("s") * NC + lax.axis_index("c")

    pltpu.sync_copy(pos_hbm, posv)

    lane = lax.iota(jnp.int32, 16)

    def chunk_body(t, carry):
        ci = wid + t * NW
        base = ci * CHUNK
        pltpu.sync_copy(src_hbm.at[pl.ds(base, CHUNK)], srcv)
        pltpu.sync_copy(dst_hbm.at[pl.ds(base, CHUNK)], dstv)
        pltpu.async_copy(feat_hbm.at[srcv], fbuf, sem).wait()

        def group_body(g, c2):
            lo = g * 16
            ev = lane + lo
            s3 = srcv[pl.ds(lo, 16)] * 3
            d3 = dstv[pl.ds(lo, 16)] * 3
            for c in range(3):
                ps = plsc.load_gather(posv, [s3 + c])
                pd = plsc.load_gather(posv, [d3 + c])
                obuf[c, pl.ds(lo, 16)] = ps - pd
            for c in range(D_FEAT):
                v = plsc.load_gather(fbuf, [ev, jnp.full((16,), c, jnp.int32)])
                obuf[3 + c, pl.ds(lo, 16)] = v
            return c2

        lax.fori_loop(0, NGROUPS, group_body, 0)

        pltpu.sync_copy(obuf, out_hbm.at[:, pl.ds(base, CHUNK)])
        return carry

    nchunks = (NCHUNKS_TOTAL - 1 - wid) // NW + 1
    lax.fori_loop(0, nchunks, chunk_body, 0)


def kernel(pos, feat, edge_index):
    ei = edge_index.astype(jnp.int32)
    mesh = plsc.VectorSubcoreMesh(core_axis_name="c", subcore_axis_name="s")

    run = functools.partial(
        pl.kernel,
        mesh=mesh,
        compiler_params=pltpu.CompilerParams(
            needs_layout_passes=False, use_tc_tiling_on_sc=True),
        out_type=jax.ShapeDtypeStruct((D_OUT, N_EDGES), jnp.float32),
        scratch_types=[
            pltpu.VMEM((3 * N_NODES,), jnp.float32),
            pltpu.VMEM((CHUNK,), jnp.int32),
            pltpu.VMEM((CHUNK,), jnp.int32),
            pltpu.VMEM((CHUNK, D_FEAT), jnp.float32),
            pltpu.VMEM((D_OUT, CHUNK), jnp.float32),
            pltpu.SemaphoreType.DMA,
        ],
    )(_sc_kernel)
    return run(pos.reshape(-1), feat, ei[0], ei[1]).T


# speedup vs baseline: 1.0965x; 1.0965x over previous
import functools

import jax
import jax.numpy as jnp
from jax import lax
from jax.experimental import pallas as pl
from jax.experimental.pallas import tpu as pltpu
from jax.experimental.pallas import tpu_sc as plsc

N_NODES = 10000
N_EDGES = 320000
D_FEAT = 128
D_OUT = D_FEAT + 3

NC = 2
NS = 16
NW = NC * NS
CHUNK = 128
NCHUNKS_TOTAL = N_EDGES // CHUNK
NGROUPS = CHUNK // 16
NCH = NCHUNKS_TOTAL // NW
REM = NCHUNKS_TOTAL - NW * NCH


def _sc_kernel(pos_hbm, feat_hbm, src_hbm, dst_hbm, out_hbm, posv,
               srcv0, dstv0, fbuf0, obuf0, srcv1, dstv1, fbuf1, obuf1,
               semi0, semi1, semg0, semg1, semo0, semo1):
    wid = lax.axis_index("s") * NC + lax.axis_index("c")
    srcv = [srcv0, srcv1]
    dstv = [dstv0, dstv1]
    fbuf = [fbuf0, fbuf1]
    obuf = [obuf0, obuf1]
    semi = [semi0, semi1]
    semg = [semg0, semg1]
    semo = [semo0, semo1]

    pltpu.sync_copy(pos_hbm, posv)

    lane = lax.iota(jnp.int32, 16)

    def base_of(t):
        return (wid + t * NW) * CHUNK

    def start_idx(t, s):
        b = base_of(t)
        pltpu.async_copy(src_hbm.at[pl.ds(b, CHUNK)], srcv[s], semi[s])
        pltpu.async_copy(dst_hbm.at[pl.ds(b, CHUNK)], dstv[s], semi[s])

    def wait_idx(t, s):
        b = base_of(t)
        pltpu.make_async_copy(
            src_hbm.at[pl.ds(b, CHUNK)], srcv[s], semi[s]).wait()
        pltpu.make_async_copy(
            dst_hbm.at[pl.ds(b, CHUNK)], dstv[s], semi[s]).wait()

    def assemble(s):
        sv, dv, fb, ob = srcv[s], dstv[s], fbuf[s], obuf[s]

        def group_body(g, c2):
            lo = g * 16
            ev = lane + lo
            s3 = sv[pl.ds(lo, 16)] * 3
            d3 = dv[pl.ds(lo, 16)] * 3
            for c in range(3):
                ps = plsc.load_gather(posv, [s3 + c])
                pd = plsc.load_gather(posv, [d3 + c])
                ob[c, pl.ds(lo, 16)] = ps - pd
            for c in range(D_FEAT):
                v = plsc.load_gather(fb, [ev, jnp.full((16,), c, jnp.int32)])
                ob[3 + c, pl.ds(lo, 16)] = v
            return c2

        lax.fori_loop(0, NGROUPS, group_body, 0)

    start_idx(0, 0)

    def pair_body(u, carry):
        for par in range(2):
            t = u * 2 + par
            s = par
            b = base_of(t)
            wait_idx(t, s)
            pltpu.async_copy(feat_hbm.at[srcv[s]], fbuf[s], semg[s])

            @pl.when(t < NCH - 1)
            def _prefetch():
                start_idx(t + 1, 1 - s)

            pltpu.make_async_copy(feat_hbm.at[srcv[s]], fbuf[s],
                                  semg[s]).wait()

            @pl.when(t >= 2)
            def _drain_prev():
                pltpu.make_async_copy(
                    obuf[s], out_hbm.at[:, pl.ds(b, CHUNK)], semo[s]).wait()

            assemble(s)
            pltpu.async_copy(obuf[s], out_hbm.at[:, pl.ds(b, CHUNK)], semo[s])
        return carry

    lax.fori_loop(0, NCH // 2, pair_body, 0)

    for s, t in ((0, NCH - 2), (1, NCH - 1)):
        pltpu.make_async_copy(
            obuf[s], out_hbm.at[:, pl.ds(base_of(t), CHUNK)], semo[s]).wait()

    @pl.when(wid < REM)
    def _remainder():
        b = (NW * NCH + wid) * CHUNK
        pltpu.sync_copy(src_hbm.at[pl.ds(b, CHUNK)], srcv0)
        pltpu.sync_copy(dst_hbm.at[pl.ds(b, CHUNK)], dstv0)
        pltpu.async_copy(feat_hbm.at[srcv0], fbuf0, semg0).wait()
        assemble(0)
        pltpu.sync_copy(obuf0, out_hbm.at[:, pl.ds(b, CHUNK)])


def kernel(pos, feat, edge_index):
    ei = edge_index.astype(jnp.int32)
    mesh = plsc.VectorSubcoreMesh(core_axis_name="c", subcore_axis_name="s")

    run = functools.partial(
        pl.kernel,
        mesh=mesh,
        compiler_params=pltpu.CompilerParams(
            needs_layout_passes=False, use_tc_tiling_on_sc=True),
        out_type=jax.ShapeDtypeStruct((D_OUT, N_EDGES), jnp.float32),
        scratch_types=[
            pltpu.VMEM((3 * N_NODES,), jnp.float32),
            pltpu.VMEM((CHUNK,), jnp.int32),
            pltpu.VMEM((CHUNK,), jnp.int32),
            pltpu.VMEM((CHUNK, D_FEAT), jnp.float32),
            pltpu.VMEM((D_OUT, CHUNK), jnp.float32),
            pltpu.VMEM((CHUNK,), jnp.int32),
            pltpu.VMEM((CHUNK,), jnp.int32),
            pltpu.VMEM((CHUNK, D_FEAT), jnp.float32),
            pltpu.VMEM((D_OUT, CHUNK), jnp.float32),
            pltpu.SemaphoreType.DMA,
            pltpu.SemaphoreType.DMA,
            pltpu.SemaphoreType.DMA,
            pltpu.SemaphoreType.DMA,
            pltpu.SemaphoreType.DMA,
            pltpu.SemaphoreType.DMA,
        ],
    )(_sc_kernel)
    return run(pos.reshape(-1), feat, ei[0], ei[1]).T


# speedup vs baseline: 1.6848x; 1.5365x over previous
import functools

import jax
import jax.numpy as jnp
from jax import lax
from jax.experimental import pallas as pl
from jax.experimental.pallas import tpu as pltpu
from jax.experimental.pallas import tpu_sc as plsc

N_NODES = 10000
N_EDGES = 320000
D_FEAT = 128
D_OUT = D_FEAT + 3

NC = 2
NS = 16
NW = NC * NS
CHUNK = 128
NCHUNKS_TOTAL = N_EDGES // CHUNK
NGROUPS = CHUNK // 16
NCH = NCHUNKS_TOTAL // NW
REM = NCHUNKS_TOTAL - NW * NCH


def _sc_kernel(pos_hbm, feat_hbm, src_hbm, dst_hbm, out_hbm, posv,
               srcv0, dstv0, fbuf0, obuf0, srcv1, dstv1, fbuf1, obuf1,
               semi0, semi1, semg0, semg1, semo0, semo1):
    wid = lax.axis_index("s") * NC + lax.axis_index("c")
    srcv = [srcv0, srcv1]
    dstv = [dstv0, dstv1]
    fbuf = [fbuf0, fbuf1]
    obuf = [obuf0, obuf1]
    semi = [semi0, semi1]
    semg = [semg0, semg1]
    semo = [semo0, semo1]

    pltpu.sync_copy(pos_hbm, posv)

    lane = lax.iota(jnp.int32, 16)

    def base_of(t):
        return (wid + t * NW) * CHUNK

    def start_idx(t, s):
        b = base_of(t)
        pltpu.async_copy(src_hbm.at[pl.ds(b, CHUNK)], srcv[s], semi[s])
        pltpu.async_copy(dst_hbm.at[pl.ds(b, CHUNK)], dstv[s], semi[s])

    def wait_idx(t, s):
        b = base_of(t)
        pltpu.make_async_copy(
            src_hbm.at[pl.ds(b, CHUNK)], srcv[s], semi[s]).wait()
        pltpu.make_async_copy(
            dst_hbm.at[pl.ds(b, CHUNK)], dstv[s], semi[s]).wait()

    def assemble(s):
        sv, dv, fb, ob = srcv[s], dstv[s], fbuf[s], obuf[s]

        def group_body(g, c2):
            lo = g * 16
            ev = lane + lo
            s3 = sv[pl.ds(lo, 16)] * 3
            d3 = dv[pl.ds(lo, 16)] * 3
            for c in range(3):
                ps = plsc.load_gather(posv, [s3 + c])
                pd = plsc.load_gather(posv, [d3 + c])
                ob[c, pl.ds(lo, 16)] = ps - pd
            for cb in range(0, D_FEAT, 8):
                vals = [
                    plsc.load_gather(
                        fb, [ev, jnp.full((16,), cb + i, jnp.int32)])
                    for i in range(8)
                ]
                for i in range(8):
                    ob[3 + cb + i, pl.ds(lo, 16)] = vals[i]
            return c2

        lax.fori_loop(0, NGROUPS, group_body, 0)

    start_idx(0, 0)

    def pair_body(u, carry):
        for par in range(2):
            t = u * 2 + par
            s = par
            b = base_of(t)
            wait_idx(t, s)
            pltpu.async_copy(feat_hbm.at[srcv[s]], fbuf[s], semg[s])

            @pl.when(t < NCH - 1)
            def _prefetch():
                start_idx(t + 1, 1 - s)

            pltpu.make_async_copy(feat_hbm.at[srcv[s]], fbuf[s],
                                  semg[s]).wait()

            @pl.when(t >= 2)
            def _drain_prev():
                pltpu.make_async_copy(
                    obuf[s], out_hbm.at[:, pl.ds(b, CHUNK)], semo[s]).wait()

            assemble(s)
            pltpu.async_copy(obuf[s], out_hbm.at[:, pl.ds(b, CHUNK)], semo[s])
        return carry

    lax.fori_loop(0, NCH // 2, pair_body, 0)

    for s, t in ((0, NCH - 2), (1, NCH - 1)):
        pltpu.make_async_copy(
            obuf[s], out_hbm.at[:, pl.ds(base_of(t), CHUNK)], semo[s]).wait()

    @pl.when(wid < REM)
    def _remainder():
        b = (NW * NCH + wid) * CHUNK
        pltpu.sync_copy(src_hbm.at[pl.ds(b, CHUNK)], srcv0)
        pltpu.sync_copy(dst_hbm.at[pl.ds(b, CHUNK)], dstv0)
        pltpu.async_copy(feat_hbm.at[srcv0], fbuf0, semg0).wait()
        assemble(0)
        pltpu.sync_copy(obuf0, out_hbm.at[:, pl.ds(b, CHUNK)])


def kernel(pos, feat, edge_index):
    ei = edge_index.astype(jnp.int32)
    mesh = plsc.VectorSubcoreMesh(core_axis_name="c", subcore_axis_name="s")

    run = functools.partial(
        pl.kernel,
        mesh=mesh,
        compiler_params=pltpu.CompilerParams(
            needs_layout_passes=False, use_tc_tiling_on_sc=True),
        out_type=jax.ShapeDtypeStruct((D_OUT, N_EDGES), jnp.float32),
        scratch_types=[
            pltpu.VMEM((3 * N_NODES,), jnp.float32),
            pltpu.VMEM((CHUNK,), jnp.int32),
            pltpu.VMEM((CHUNK,), jnp.int32),
            pltpu.VMEM((CHUNK, D_FEAT), jnp.float32),
            pltpu.VMEM((D_OUT, CHUNK), jnp.float32),
            pltpu.VMEM((CHUNK,), jnp.int32),
            pltpu.VMEM((CHUNK,), jnp.int32),
            pltpu.VMEM((CHUNK, D_FEAT), jnp.float32),
            pltpu.VMEM((D_OUT, CHUNK), jnp.float32),
            pltpu.SemaphoreType.DMA,
            pltpu.SemaphoreType.DMA,
            pltpu.SemaphoreType.DMA,
            pltpu.SemaphoreType.DMA,
            pltpu.SemaphoreType.DMA,
            pltpu.SemaphoreType.DMA,
        ],
    )(_sc_kernel)
    return run(pos.reshape(-1), feat, ei[0], ei[1]).T


# speedup vs baseline: 1.7130x; 1.0167x over previous
import functools

import jax
import jax.numpy as jnp
from jax import lax
from jax.experimental import pallas as pl
from jax.experimental.pallas import tpu as pltpu
from jax.experimental.pallas import tpu_sc as plsc

N_NODES = 10000
N_EDGES = 320000
D_FEAT = 128
D_OUT = D_FEAT + 3

NC = 2
NS = 16
NW = NC * NS
CHUNK = 128
NCHUNKS_TOTAL = N_EDGES // CHUNK
NGROUPS = CHUNK // 16
NCH = NCHUNKS_TOTAL // NW
REM = NCHUNKS_TOTAL - NW * NCH


def _sc_kernel(pos_hbm, feat_hbm, src_hbm, dst_hbm, out_hbm, posv,
               srcv0, dstv0, fbuf0, obuf0, srcv1, dstv1, fbuf1, obuf1,
               semi0, semi1, semg0, semg1, semo0, semo1):
    wid = lax.axis_index("s") * NC + lax.axis_index("c")
    srcv = [srcv0, srcv1]
    dstv = [dstv0, dstv1]
    fbuf = [fbuf0, fbuf1]
    obuf = [obuf0, obuf1]
    semi = [semi0, semi1]
    semg = [semg0, semg1]
    semo = [semo0, semo1]

    pltpu.sync_copy(pos_hbm, posv)

    lane = lax.iota(jnp.int32, 16)

    def base_of(t):
        return (wid + t * NW) * CHUNK

    def start_idx(t, s):
        b = base_of(t)
        pltpu.async_copy(src_hbm.at[pl.ds(b, CHUNK)], srcv[s], semi[s])
        pltpu.async_copy(dst_hbm.at[pl.ds(b, CHUNK)], dstv[s], semi[s])

    def wait_idx(t, s):
        b = base_of(t)
        pltpu.make_async_copy(
            src_hbm.at[pl.ds(b, CHUNK)], srcv[s], semi[s]).wait()
        pltpu.make_async_copy(
            dst_hbm.at[pl.ds(b, CHUNK)], dstv[s], semi[s]).wait()

    def assemble(s):
        sv, dv, fb, ob = srcv[s], dstv[s], fbuf[s], obuf[s]

        def group_body(g, c2):
            lo = g * 16
            ev = lane + lo
            s3 = sv[pl.ds(lo, 16)] * 3
            d3 = dv[pl.ds(lo, 16)] * 3
            for c in range(3):
                ps = plsc.load_gather(posv, [s3 + c])
                pd = plsc.load_gather(posv, [d3 + c])
                ob[c, pl.ds(lo, 16)] = ps - pd
            for cb in range(0, D_FEAT, 16):
                vals = [
                    plsc.load_gather(
                        fb, [ev, jnp.full((16,), cb + i, jnp.int32)])
                    for i in range(16)
                ]
                for i in range(16):
                    ob[3 + cb + i, pl.ds(lo, 16)] = vals[i]
            return c2

        lax.fori_loop(0, NGROUPS, group_body, 0)

    start_idx(0, 0)

    def pair_body(u, carry):
        for par in range(2):
            t = u * 2 + par
            s = par
            b = base_of(t)
            wait_idx(t, s)
            pltpu.async_copy(feat_hbm.at[srcv[s]], fbuf[s], semg[s])

            @pl.when(t < NCH - 1)
            def _prefetch():
                start_idx(t + 1, 1 - s)

            pltpu.make_async_copy(feat_hbm.at[srcv[s]], fbuf[s],
                                  semg[s]).wait()

            @pl.when(t >= 2)
            def _drain_prev():
                pltpu.make_async_copy(
                    obuf[s], out_hbm.at[:, pl.ds(b, CHUNK)], semo[s]).wait()

            assemble(s)
            pltpu.async_copy(obuf[s], out_hbm.at[:, pl.ds(b, CHUNK)], semo[s])
        return carry

    lax.fori_loop(0, NCH // 2, pair_body, 0)

    for s, t in ((0, NCH - 2), (1, NCH - 1)):
        pltpu.make_async_copy(
            obuf[s], out_hbm.at[:, pl.ds(base_of(t), CHUNK)], semo[s]).wait()

    @pl.when(wid < REM)
    def _remainder():
        b = (NW * NCH + wid) * CHUNK
        pltpu.sync_copy(src_hbm.at[pl.ds(b, CHUNK)], srcv0)
        pltpu.sync_copy(dst_hbm.at[pl.ds(b, CHUNK)], dstv0)
        pltpu.async_copy(feat_hbm.at[srcv0], fbuf0, semg0).wait()
        assemble(0)
        pltpu.sync_copy(obuf0, out_hbm.at[:, pl.ds(b, CHUNK)])


def kernel(pos, feat, edge_index):
    ei = edge_index.astype(jnp.int32)
    mesh = plsc.VectorSubcoreMesh(core_axis_name="c", subcore_axis_name="s")

    run = functools.partial(
        pl.kernel,
        mesh=mesh,
        compiler_params=pltpu.CompilerParams(
            needs_layout_passes=False, use_tc_tiling_on_sc=True),
        out_type=jax.ShapeDtypeStruct((D_OUT, N_EDGES), jnp.float32),
        scratch_types=[
            pltpu.VMEM((3 * N_NODES,), jnp.float32),
            pltpu.VMEM((CHUNK,), jnp.int32),
            pltpu.VMEM((CHUNK,), jnp.int32),
            pltpu.VMEM((CHUNK, D_FEAT), jnp.float32),
            pltpu.VMEM((D_OUT, CHUNK), jnp.float32),
            pltpu.VMEM((CHUNK,), jnp.int32),
            pltpu.VMEM((CHUNK,), jnp.int32),
            pltpu.VMEM((CHUNK, D_FEAT), jnp.float32),
            pltpu.VMEM((D_OUT, CHUNK), jnp.float32),
            pltpu.SemaphoreType.DMA,
            pltpu.SemaphoreType.DMA,
            pltpu.SemaphoreType.DMA,
            pltpu.SemaphoreType.DMA,
            pltpu.SemaphoreType.DMA,
            pltpu.SemaphoreType.DMA,
        ],
    )(_sc_kernel)
    return run(pos.reshape(-1), feat, ei[0], ei[1]).T


# speedup vs baseline: 1.9119x; 1.1161x over previous
import functools

import jax
import jax.numpy as jnp
from jax import lax
from jax.experimental import pallas as pl
from jax.experimental.pallas import tpu as pltpu
from jax.experimental.pallas import tpu_sc as plsc

N_NODES = 10000
N_EDGES = 320000
D_FEAT = 128
D_PAD = 128
D_OUT = D_FEAT + 3

NC = 2
NS = 16
NW = NC * NS
CHUNK = 128
NCHUNKS_TOTAL = N_EDGES // CHUNK
NGROUPS = CHUNK // 16
NCH = NCHUNKS_TOTAL // NW
REM = NCHUNKS_TOTAL - NW * NCH


def _sc_kernel(pos_hbm, feat_hbm, src_hbm, dst_hbm, out_hbm, posv,
               srcv0, dstv0, fbuf0, obuf0, srcv1, dstv1, fbuf1, obuf1,
               semi0, semi1, semg0, semg1, semo0, semo1):
    wid = lax.axis_index("s") * NC + lax.axis_index("c")
    srcv = [srcv0, srcv1]
    dstv = [dstv0, dstv1]
    fbuf = [fbuf0, fbuf1]
    obuf = [obuf0, obuf1]
    semi = [semi0, semi1]
    semg = [semg0, semg1]
    semo = [semo0, semo1]

    pltpu.sync_copy(pos_hbm, posv)

    lane = lax.iota(jnp.int32, 16)

    def base_of(t):
        return (wid + t * NW) * CHUNK

    def start_idx(t, s):
        b = base_of(t)
        pltpu.async_copy(src_hbm.at[pl.ds(b, CHUNK)], srcv[s], semi[s])
        pltpu.async_copy(dst_hbm.at[pl.ds(b, CHUNK)], dstv[s], semi[s])

    def wait_idx(t, s):
        b = base_of(t)
        pltpu.make_async_copy(
            src_hbm.at[pl.ds(b, CHUNK)], srcv[s], semi[s]).wait()
        pltpu.make_async_copy(
            dst_hbm.at[pl.ds(b, CHUNK)], dstv[s], semi[s]).wait()

    def assemble(s):
        sv, dv, fb, ob = srcv[s], dstv[s], fbuf[s], obuf[s]

        def group_body(g, c2):
            lo = g * 16
            ev = lane + lo
            s3 = sv[pl.ds(lo, 16)] * 3
            d3 = dv[pl.ds(lo, 16)] * 3
            for c in range(3):
                ps = plsc.load_gather(posv, [s3 + c])
                pd = plsc.load_gather(posv, [d3 + c])
                ob[c, pl.ds(lo, 16)] = ps - pd
            for cb in range(0, D_FEAT, 16):
                vals = [
                    plsc.load_gather(
                        fb, [ev, jnp.full((16,), cb + i, jnp.int32)])
                    for i in range(16)
                ]
                for i in range(16):
                    ob[3 + cb + i, pl.ds(lo, 16)] = vals[i]
            return c2

        lax.fori_loop(0, NGROUPS, group_body, 0)

    start_idx(0, 0)
    wait_idx(0, 0)
    pltpu.async_copy(feat_hbm.at[srcv[0]], fbuf[0], semg[0])
    start_idx(1, 1)

    def pair_body(u, carry):
        for par in range(2):
            t = u * 2 + par
            s = par
            b = base_of(t)

            @pl.when(t < NCH - 1)
            def _issue_next_gather():
                wait_idx(t + 1, 1 - s)
                pltpu.async_copy(feat_hbm.at[srcv[1 - s]], fbuf[1 - s],
                                 semg[1 - s])

            pltpu.make_async_copy(feat_hbm.at[srcv[s]], fbuf[s],
                                  semg[s]).wait()

            @pl.when(t >= 2)
            def _drain_prev():
                pltpu.make_async_copy(
                    obuf[s], out_hbm.at[:, pl.ds(b, CHUNK)], semo[s]).wait()

            assemble(s)
            pltpu.async_copy(obuf[s], out_hbm.at[:, pl.ds(b, CHUNK)], semo[s])

            @pl.when(t < NCH - 2)
            def _prefetch_idx():
                start_idx(t + 2, s)
        return carry

    lax.fori_loop(0, NCH // 2, pair_body, 0)

    for s, t in ((0, NCH - 2), (1, NCH - 1)):
        pltpu.make_async_copy(
            obuf[s], out_hbm.at[:, pl.ds(base_of(t), CHUNK)], semo[s]).wait()

    @pl.when(wid < REM)
    def _remainder():
        b = (NW * NCH + wid) * CHUNK
        pltpu.sync_copy(src_hbm.at[pl.ds(b, CHUNK)], srcv0)
        pltpu.sync_copy(dst_hbm.at[pl.ds(b, CHUNK)], dstv0)
        pltpu.async_copy(feat_hbm.at[srcv0], fbuf0, semg0).wait()
        assemble(0)
        pltpu.sync_copy(obuf0, out_hbm.at[:, pl.ds(b, CHUNK)])


def kernel(pos, feat, edge_index):
    ei = edge_index.astype(jnp.int32)
    featp = jnp.pad(feat, ((0, 0), (0, D_PAD - D_FEAT)))
    mesh = plsc.VectorSubcoreMesh(core_axis_name="c", subcore_axis_name="s")

    run = functools.partial(
        pl.kernel,
        mesh=mesh,
        compiler_params=pltpu.CompilerParams(
            needs_layout_passes=False, use_tc_tiling_on_sc=True),
        out_type=jax.ShapeDtypeStruct((D_OUT, N_EDGES), jnp.float32),
        scratch_types=[
            pltpu.VMEM((3 * N_NODES,), jnp.float32),
            pltpu.VMEM((CHUNK,), jnp.int32),
            pltpu.VMEM((CHUNK,), jnp.int32),
            pltpu.VMEM((CHUNK, D_PAD), jnp.float32),
            pltpu.VMEM((D_OUT, CHUNK), jnp.float32),
            pltpu.VMEM((CHUNK,), jnp.int32),
            pltpu.VMEM((CHUNK,), jnp.int32),
            pltpu.VMEM((CHUNK, D_PAD), jnp.float32),
            pltpu.VMEM((D_OUT, CHUNK), jnp.float32),
            pltpu.SemaphoreType.DMA,
            pltpu.SemaphoreType.DMA,
            pltpu.SemaphoreType.DMA,
            pltpu.SemaphoreType.DMA,
            pltpu.SemaphoreType.DMA,
            pltpu.SemaphoreType.DMA,
        ],
    )(_sc_kernel)
    return run(pos.reshape(-1), featp, ei[0], ei[1]).T


# speedup vs baseline: 4.5593x; 2.3847x over previous
import functools

import jax
import jax.numpy as jnp
from jax import lax
from jax.experimental import pallas as pl
from jax.experimental.pallas import tpu as pltpu
from jax.experimental.pallas import tpu_sc as plsc

N_NODES = 10000
N_EDGES = 320000
D_FEAT = 128
D_PAD = 128
D_OUT = D_FEAT + 3

NC = 2
NS = 16
NW = NC * NS
CHUNK = 128
NCHUNKS_TOTAL = N_EDGES // CHUNK
NGROUPS = CHUNK // 16
NCH = NCHUNKS_TOTAL // NW
REM = NCHUNKS_TOTAL - NW * NCH


def _sc_kernel(pos_hbm, feat_hbm, src_hbm, dst_hbm, out_hbm, posv,
               srcv0, dstv0, fbuf0, obuf0, srcv1, dstv1, fbuf1, obuf1,
               semi0, semi1, semg0, semg1, semo0, semo1):
    wid = lax.axis_index("s") * NC + lax.axis_index("c")
    srcv = [srcv0, srcv1]
    dstv = [dstv0, dstv1]
    fbuf = [fbuf0, fbuf1]
    obuf = [obuf0, obuf1]
    semi = [semi0, semi1]
    semg = [semg0, semg1]
    semo = [semo0, semo1]

    pltpu.sync_copy(pos_hbm, posv)

    lane = lax.iota(jnp.int32, 16)

    def base_of(t):
        return (wid + t * NW) * CHUNK

    def start_idx(t, s):
        b = base_of(t)
        pltpu.async_copy(src_hbm.at[pl.ds(b, CHUNK)], srcv[s], semi[s])
        pltpu.async_copy(dst_hbm.at[pl.ds(b, CHUNK)], dstv[s], semi[s])

    def wait_idx(t, s):
        b = base_of(t)
        pltpu.make_async_copy(
            src_hbm.at[pl.ds(b, CHUNK)], srcv[s], semi[s]).wait()
        pltpu.make_async_copy(
            dst_hbm.at[pl.ds(b, CHUNK)], dstv[s], semi[s]).wait()

    def assemble(s):
        sv, dv, fb, ob = srcv[s], dstv[s], fbuf[s], obuf[s]

        def group_body(g, c2):
            lo = g * 16
            ev = lane + lo
            s3 = sv[pl.ds(lo, 16)] * 3
            d3 = dv[pl.ds(lo, 16)] * 3
            for c in range(3):
                ps = plsc.load_gather(posv, [s3 + c])
                pd = plsc.load_gather(posv, [d3 + c])
                ob[c, pl.ds(lo, 16)] = ps - pd
            for cb in range(0, D_FEAT, 16):
                pairs = []
                for j in range(16):
                    rot = (lane + j) % 16
                    cvec = rot + cb
                    v = plsc.load_gather(fb, [ev, cvec])
                    pairs.append((cvec + 3, v))
                for rvec, v in pairs:
                    plsc.store_scatter(ob, [rvec, ev], v)
            return c2

        lax.fori_loop(0, NGROUPS, group_body, 0)

    start_idx(0, 0)
    wait_idx(0, 0)
    pltpu.async_copy(feat_hbm.at[srcv[0]], fbuf[0], semg[0])
    start_idx(1, 1)

    def pair_body(u, carry):
        for par in range(2):
            t = u * 2 + par
            s = par
            b = base_of(t)

            @pl.when(t < NCH - 1)
            def _issue_next_gather():
                wait_idx(t + 1, 1 - s)
                pltpu.async_copy(feat_hbm.at[srcv[1 - s]], fbuf[1 - s],
                                 semg[1 - s])

            pltpu.make_async_copy(feat_hbm.at[srcv[s]], fbuf[s],
                                  semg[s]).wait()

            @pl.when(t >= 2)
            def _drain_prev():
                pltpu.make_async_copy(
                    obuf[s], out_hbm.at[:, pl.ds(b, CHUNK)], semo[s]).wait()

            assemble(s)
            pltpu.async_copy(obuf[s], out_hbm.at[:, pl.ds(b, CHUNK)], semo[s])

            @pl.when(t < NCH - 2)
            def _prefetch_idx():
                start_idx(t + 2, s)
        return carry

    lax.fori_loop(0, NCH // 2, pair_body, 0)

    for s, t in ((0, NCH - 2), (1, NCH - 1)):
        pltpu.make_async_copy(
            obuf[s], out_hbm.at[:, pl.ds(base_of(t), CHUNK)], semo[s]).wait()

    @pl.when(wid < REM)
    def _remainder():
        b = (NW * NCH + wid) * CHUNK
        pltpu.sync_copy(src_hbm.at[pl.ds(b, CHUNK)], srcv0)
        pltpu.sync_copy(dst_hbm.at[pl.ds(b, CHUNK)], dstv0)
        pltpu.async_copy(feat_hbm.at[srcv0], fbuf0, semg0).wait()
        assemble(0)
        pltpu.sync_copy(obuf0, out_hbm.at[:, pl.ds(b, CHUNK)])


def kernel(pos, feat, edge_index):
    ei = edge_index.astype(jnp.int32)
    featp = jnp.pad(feat, ((0, 0), (0, D_PAD - D_FEAT)))
    mesh = plsc.VectorSubcoreMesh(core_axis_name="c", subcore_axis_name="s")

    run = functools.partial(
        pl.kernel,
        mesh=mesh,
        compiler_params=pltpu.CompilerParams(
            needs_layout_passes=False, use_tc_tiling_on_sc=True),
        out_type=jax.ShapeDtypeStruct((D_OUT, N_EDGES), jnp.float32),
        scratch_types=[
            pltpu.VMEM((3 * N_NODES,), jnp.float32),
            pltpu.VMEM((CHUNK,), jnp.int32),
            pltpu.VMEM((CHUNK,), jnp.int32),
            pltpu.VMEM((CHUNK, D_PAD), jnp.float32),
            pltpu.VMEM((D_OUT, CHUNK), jnp.float32),
            pltpu.VMEM((CHUNK,), jnp.int32),
            pltpu.VMEM((CHUNK,), jnp.int32),
            pltpu.VMEM((CHUNK, D_PAD), jnp.float32),
            pltpu.VMEM((D_OUT, CHUNK), jnp.float32),
            pltpu.SemaphoreType.DMA,
            pltpu.SemaphoreType.DMA,
            pltpu.SemaphoreType.DMA,
            pltpu.SemaphoreType.DMA,
            pltpu.SemaphoreType.DMA,
            pltpu.SemaphoreType.DMA,
        ],
    )(_sc_kernel)
    return run(pos.reshape(-1), featp, ei[0], ei[1]).T
